# baseline (speedup 1.0000x reference)
import functools

import jax
import jax.numpy as jnp
from jax import lax
from jax.experimental import pallas as pl
from jax.experimental.pallas import tpu as pltpu
from jax.experimental.pallas import tpu_sc as plsc


@functools.cache
def _make_gather(B, V, D):
    info = plsc.get_sparse_core_info()
    NC, NS = info.num_cores, info.num_subcores
    NW = NC * NS
    assert B % NW == 0
    b_per_w = B // NW
    C = 512
    NBUF = 2
    assert b_per_w % (C * NBUF) == 0
    n_chunks = b_per_w // C
    n_groups = n_chunks // NBUF

    mesh = plsc.VectorSubcoreMesh(core_axis_name="c", subcore_axis_name="s")

    @functools.partial(
        pl.kernel,
        mesh=mesh,
        compiler_params=pltpu.CompilerParams(use_tc_tiling_on_sc=False),
        out_type=jax.ShapeDtypeStruct((B, D), jnp.float32),
        scratch_types=[
            pltpu.VMEM((b_per_w,), jnp.int32),
            pltpu.VMEM((NBUF, C, D), jnp.float32),
            [pltpu.SemaphoreType.DMA] * NBUF,
            [pltpu.SemaphoreType.DMA] * NBUF,
        ],
    )
    def gather_kernel(idx_hbm, table_hbm, out_hbm, idx_v, rows_v, gsems, osems):
        wid = lax.axis_index("s") * NC + lax.axis_index("c")
        base = wid * b_per_w

        pltpu.sync_copy(idx_hbm.at[pl.ds(base, b_per_w)], idx_v)

        def gather_dma(i, b):
            return pltpu.make_async_copy(
                table_hbm.at[idx_v.at[pl.ds(i * C, C)]], rows_v.at[b], gsems[b]
            )

        def store_dma(i, b):
            return pltpu.make_async_copy(
                rows_v.at[b], out_hbm.at[pl.ds(base + i * C, C)], osems[b]
            )

        for b in range(NBUF):
            gather_dma(b, b).start()

        def group(g, carry):
            for b in range(NBUF):
                i = g * NBUF + b
                gather_dma(i, b).wait()
                store_dma(i, b).start()
            for b in range(NBUF):
                i = g * NBUF + b
                store_dma(i, b).wait()
                gather_dma(i + NBUF, b).start()
            return carry

        lax.fori_loop(0, n_groups - 1, group, 0)

        for b in range(NBUF):
            i = (n_groups - 1) * NBUF + b
            gather_dma(i, b).wait()
            store_dma(i, b).start()
        for b in range(NBUF):
            i = (n_groups - 1) * NBUF + b
            store_dma(i, b).wait()

    return gather_kernel


_BC = 1024


def _relayout_body(x_ref, o_ref):
    xt = x_ref[...].T
    o_ref[...] = jnp.concatenate([xt[: _BC // 2], xt[_BC // 2 :]], axis=1)


@functools.cache
def _make_relayout(V, D):
    nblk = pl.cdiv(V, _BC)
    return pl.pallas_call(
        _relayout_body,
        grid=(nblk,),
        in_specs=[pl.BlockSpec((D, _BC), lambda c: (0, c))],
        out_specs=pl.BlockSpec((_BC // 2, 2 * D), lambda c: (c, 0)),
        out_shape=jax.ShapeDtypeStruct((nblk * _BC // 2, 2 * D), jnp.float32),
    )


def kernel(goal_encoding, embed_table):
    batch, hist = goal_encoding.shape
    v, d = embed_table.shape
    i = goal_encoding.reshape(-1).astype(jnp.int32)
    idx = (i & ~(_BC - 1)) + ((i & (_BC // 2 - 1)) << 1) + ((i >> 9) & 1)
    lin = _make_relayout(v, d)(embed_table.T)
    v2 = 2 * lin.shape[0]
    table_lin = lin.reshape(v2, d)
    out = _make_gather(batch * hist, v2, d)(idx, table_lin)
    return out.reshape(batch, hist, d)
